# Initial kernel scaffold; baseline (speedup 1.0000x reference)
import jax
import jax.numpy as jnp
from jax.experimental import pallas as pl


def kernel(x, table):
    raise NotImplementedError("write your pallas kernel here")


# speedup vs baseline: 1.0226x; 1.0226x over previous
import functools

import jax
import jax.numpy as jnp
from jax import lax
from jax.experimental import pallas as pl
from jax.experimental.pallas import tpu as pltpu
from jax.experimental.pallas import tpu_sc as plsc

EMBED_DIM = 32
NUM_CORES = 2
NUM_SUBCORES = 16
NW = NUM_CORES * NUM_SUBCORES
CHUNK = 128


@functools.lru_cache(maxsize=None)
def _make_kernel(n_idx: int):
    per_w = n_idx // NW
    n_chunks = per_w // CHUNK
    mesh = plsc.VectorSubcoreMesh(core_axis_name="c", subcore_axis_name="s")

    @functools.partial(
        pl.kernel,
        mesh=mesh,
        compiler_params=pltpu.CompilerParams(use_tc_tiling_on_sc=False),
        out_type=jax.ShapeDtypeStruct((n_idx, EMBED_DIM), jnp.float32),
        scratch_types=[
            pltpu.VMEM((n_chunks, CHUNK), jnp.int32),
            pltpu.VMEM((CHUNK, EMBED_DIM), jnp.float32),
            pltpu.SemaphoreType.DMA,
        ],
    )
    def emb(x_hbm, table_hbm, out_hbm, idx_v, rows_v, sem):
        wid = lax.axis_index("s") * NUM_CORES + lax.axis_index("c")
        base = wid * per_w
        pltpu.sync_copy(x_hbm.at[wid], idx_v)

        def body(j, carry):
            pltpu.async_copy(table_hbm.at[idx_v.at[j]], rows_v, sem).wait()
            off = pl.multiple_of(base + j * CHUNK, CHUNK)
            pltpu.sync_copy(rows_v, out_hbm.at[pl.ds(off, CHUNK)])
            return carry

        lax.fori_loop(0, n_chunks, body, 0)

    return emb


def kernel(x, table):
    b, h = x.shape
    n = b * h
    xr = x.astype(jnp.int32).reshape(NW, n // NW // CHUNK, CHUNK)
    out = _make_kernel(n)(xr, table)
    return out.reshape(b, h, EMBED_DIM)


# speedup vs baseline: 1.2854x; 1.2571x over previous
import functools

import jax
import jax.numpy as jnp
from jax import lax
from jax.experimental import pallas as pl
from jax.experimental.pallas import tpu as pltpu
from jax.experimental.pallas import tpu_sc as plsc

EMBED_DIM = 32
NUM_CORES = 2
NUM_SUBCORES = 16
NW = NUM_CORES * NUM_SUBCORES
CHUNK = 128
K = 10
BLOCK = K * CHUNK


@functools.lru_cache(maxsize=None)
def _make_kernel(n_idx: int):
    per_w = n_idx // NW
    n_chunks = per_w // CHUNK
    n_blocks = per_w // BLOCK
    mesh = plsc.VectorSubcoreMesh(core_axis_name="c", subcore_axis_name="s")

    @functools.partial(
        pl.kernel,
        mesh=mesh,
        compiler_params=pltpu.CompilerParams(use_tc_tiling_on_sc=False),
        out_type=jax.ShapeDtypeStruct((NW * n_blocks, BLOCK, EMBED_DIM),
                                      jnp.float32),
        scratch_types=[
            pltpu.VMEM((n_chunks, CHUNK), jnp.int32),
            pltpu.VMEM((2, BLOCK, EMBED_DIM), jnp.float32),
            pltpu.SemaphoreType.DMA,
            pltpu.SemaphoreType.DMA,
        ],
    )
    def emb(x_hbm, table_hbm, out_hbm, idx_v, rows_v, gsem, wsem):
        wid = lax.axis_index("s") * NUM_CORES + lax.axis_index("c")
        pltpu.sync_copy(x_hbm.at[wid], idx_v)

        def gather(tb, s, b):
            return pltpu.make_async_copy(
                table_hbm.at[idx_v.at[tb * K + b]],
                rows_v.at[s].at[pl.ds(b * CHUNK, CHUNK)],
                gsem,
            )

        def write(tb, s):
            return pltpu.make_async_copy(
                rows_v.at[s], out_hbm.at[wid * n_blocks + tb], wsem)

        for b in range(K):
            gather(0, 0, b).start()

        def body(tb, carry):
            s = lax.rem(tb, 2)
            for b in range(K):
                gather(tb, s, b).wait()
            @pl.when(tb >= 1)
            def _():
                write(tb - 1, 1 - s).wait()
            write(tb, s).start()
            @pl.when(tb + 1 < n_blocks)
            def _():
                for b in range(K):
                    gather(tb + 1, 1 - s, b).start()
            return carry

        lax.fori_loop(0, n_blocks, body, 0)
        write(n_blocks - 1, (n_blocks - 1) % 2).wait()

    return emb


def kernel(x, table):
    b, h = x.shape
    n = b * h
    xr = x.astype(jnp.int32).reshape(NW, n // NW // CHUNK, CHUNK)
    out = _make_kernel(n)(xr, table)
    return out.reshape(b, h, EMBED_DIM)
